# baseline (speedup 1.0000x reference)
import functools

import jax
import jax.numpy as jnp
from jax import lax
from jax.experimental import pallas as pl
from jax.experimental.pallas import tpu as pltpu
from jax.experimental.pallas import tpu_sc as plsc

D_MODEL = 64
SCALE = 8.0
NUM_WORKERS = 32
CHUNK = 128
NSLOT = 2


@functools.partial(jax.jit, static_argnames=("n_chunks",))
def _embed_lookup(x_flat, lut, *, n_chunks):
    chunks_per_w = n_chunks // NUM_WORKERS
    mesh = plsc.VectorSubcoreMesh(core_axis_name="c", subcore_axis_name="s")

    @functools.partial(
        pl.kernel,
        out_type=jax.ShapeDtypeStruct((200, 8, 128, 8, 128), jnp.float32),
        mesh=mesh,
        scratch_types=[
            [pltpu.VMEM((CHUNK,), jnp.int32) for _ in range(NSLOT)],
            [pltpu.VMEM((CHUNK, D_MODEL), jnp.float32) for _ in range(NSLOT)],
            [pltpu.VMEM((D_MODEL, CHUNK + 1), jnp.float32) for _ in range(NSLOT)],
            [pltpu.SemaphoreType.DMA for _ in range(NSLOT)],
            [pltpu.SemaphoreType.DMA for _ in range(NSLOT)],
            [pltpu.SemaphoreType.DMA for _ in range(NSLOT)],
        ],
        compiler_params=pltpu.CompilerParams(
            use_tc_tiling_on_sc=False, needs_layout_passes=False
        ),
    )
    def k(x_hbm, lut_hbm, out_hbm, idx, rows, tbuf, isem, gsem, wsem):
        wid = lax.axis_index("s") * 2 + lax.axis_index("c")
        base = wid * chunks_per_w
        lane = lax.broadcasted_iota(jnp.int32, (16,), 0)

        def fetch_idx(i, sl):
            c = base + i
            pltpu.async_copy(x_hbm.at[pl.ds(c * CHUNK, CHUNK)], idx[sl], isem[sl])

        def launch_gather(i, sl):
            c = base + i
            pltpu.make_async_copy(
                x_hbm.at[pl.ds(c * CHUNK, CHUNK)], idx[sl], isem[sl]
            ).wait()
            pltpu.async_copy(lut_hbm.at[idx[sl]], rows[sl], gsem[sl])

        def out_tiles(i, sl):
            c = base + i
            s = (c >> 10) * 8 + (c & 7)
            bt = (c >> 3) & 127
            return [
                (tbuf[sl].at[pl.ds(dg * 8, 8), pl.ds(0, CHUNK)], out_hbm.at[s, dg, bt])
                for dg in range(8)
            ]

        def drain_writes(i, sl):
            for src, dst in out_tiles(i, sl):
                pltpu.make_async_copy(src, dst, wsem[sl]).wait()

        def finish(i, sl):
            pltpu.make_async_copy(lut_hbm.at[idx[sl]], rows[sl], gsem[sl]).wait()

            @pl.loop(0, CHUNK, unroll=16)
            def _(r):
                rv = jnp.full((16,), 0, jnp.int32) + r
                for j in range(D_MODEL // 16):
                    v = rows[sl][r, pl.ds(j * 16, 16)]
                    plsc.store_scatter(tbuf[sl], [j * 16 + lane, rv], v * SCALE)

            for src, dst in out_tiles(i, sl):
                pltpu.async_copy(src, dst, wsem[sl])

        fetch_idx(0, 0)
        fetch_idx(1, 1)
        launch_gather(0, 0)

        @pl.loop(0, chunks_per_w, step=NSLOT)
        def _(g):
            for b in range(NSLOT):
                i = g + b
                o = 1 - b

                @pl.when(i + 1 < chunks_per_w)
                def _():
                    launch_gather(i + 1, o)

                @pl.when(i >= NSLOT)
                def _():
                    drain_writes(i - NSLOT, b)

                finish(i, b)

                @pl.when(i + 2 < chunks_per_w)
                def _():
                    fetch_idx(i + 2, b)

        for b in range(NSLOT):
            drain_writes(chunks_per_w - NSLOT + b, b)

    return k(x_flat, lut)


def kernel(x, lut):
    bsz, seq = x.shape
    vocab, d = lut.shape
    x_flat = (
        x.T.reshape(seq // 8, 8, bsz // 128, 128)
        .transpose(0, 2, 1, 3)
        .reshape(-1)
        .astype(jnp.int32)
    )
    out5 = _embed_lookup(x_flat, lut, n_chunks=(bsz * seq) // CHUNK)
    return out5.transpose(2, 4, 0, 1, 3).reshape(bsz, seq, d)


# speedup vs baseline: 1.1542x; 1.1542x over previous
import functools

import jax
import jax.numpy as jnp
from jax import lax
from jax.experimental import pallas as pl
from jax.experimental.pallas import tpu as pltpu
from jax.experimental.pallas import tpu_sc as plsc

D_MODEL = 64
SCALE = 8.0
NUM_WORKERS = 32
CHUNK = 128
NSLOT = 4


@functools.partial(jax.jit, static_argnames=("n_chunks",))
def _embed_lookup(x_flat, lut, *, n_chunks):
    chunks_per_w = n_chunks // NUM_WORKERS
    mesh = plsc.VectorSubcoreMesh(core_axis_name="c", subcore_axis_name="s")

    @functools.partial(
        pl.kernel,
        out_type=jax.ShapeDtypeStruct((200, 8, 128, 8, 128), jnp.float32),
        mesh=mesh,
        scratch_types=[
            [pltpu.VMEM((CHUNK,), jnp.int32) for _ in range(NSLOT)],
            [pltpu.VMEM((CHUNK, D_MODEL), jnp.float32) for _ in range(NSLOT)],
            [pltpu.VMEM((D_MODEL, CHUNK + 1), jnp.float32) for _ in range(NSLOT)],
            [pltpu.SemaphoreType.DMA for _ in range(NSLOT)],
            [pltpu.SemaphoreType.DMA for _ in range(NSLOT)],
            [pltpu.SemaphoreType.DMA for _ in range(NSLOT)],
        ],
        compiler_params=pltpu.CompilerParams(
            use_tc_tiling_on_sc=False, needs_layout_passes=False
        ),
    )
    def k(x_hbm, lut_hbm, out_hbm, idx, rows, tbuf, isem, gsem, wsem):
        wid = lax.axis_index("s") * 2 + lax.axis_index("c")
        base = wid * chunks_per_w
        lane = lax.broadcasted_iota(jnp.int32, (16,), 0)

        def fetch_idx(i, sl):
            c = base + i
            pltpu.async_copy(x_hbm.at[pl.ds(c * CHUNK, CHUNK)], idx[sl], isem[sl])

        def launch_gather(i, sl):
            c = base + i
            pltpu.make_async_copy(
                x_hbm.at[pl.ds(c * CHUNK, CHUNK)], idx[sl], isem[sl]
            ).wait()
            pltpu.async_copy(lut_hbm.at[idx[sl]], rows[sl], gsem[sl])

        def out_tiles(i, sl):
            c = base + i
            s = (c >> 10) * 8 + (c & 7)
            bt = (c >> 3) & 127
            return [
                (tbuf[sl].at[pl.ds(dg * 8, 8), pl.ds(0, CHUNK)], out_hbm.at[s, dg, bt])
                for dg in range(8)
            ]

        def drain_writes(i, sl):
            for src, dst in out_tiles(i, sl):
                pltpu.make_async_copy(src, dst, wsem[sl]).wait()

        def finish(i, sl):
            pltpu.make_async_copy(lut_hbm.at[idx[sl]], rows[sl], gsem[sl]).wait()

            @pl.loop(0, CHUNK, unroll=8)
            def _(r):
                rv = jnp.full((16,), 0, jnp.int32) + r
                for j in range(D_MODEL // 16):
                    v = rows[sl][r, pl.ds(j * 16, 16)]
                    plsc.store_scatter(tbuf[sl], [j * 16 + lane, rv], v * SCALE)

            for src, dst in out_tiles(i, sl):
                pltpu.async_copy(src, dst, wsem[sl])

        fetch_idx(0, 0)
        fetch_idx(1, 1)
        fetch_idx(2, 2)
        launch_gather(0, 0)
        launch_gather(1, 1)

        @pl.loop(0, chunks_per_w, step=NSLOT)
        def _(g):
            for b in range(NSLOT):
                i = g + b
                s2 = (b + 2) % NSLOT
                s3 = (b + 3) % NSLOT

                @pl.when(i + 3 < chunks_per_w)
                def _():
                    fetch_idx(i + 3, s3)

                @pl.when(i + 2 < chunks_per_w)
                def _():
                    launch_gather(i + 2, s2)

                @pl.when(i >= NSLOT)
                def _():
                    drain_writes(i - NSLOT, b)

                finish(i, b)

        for b in range(NSLOT):
            drain_writes(chunks_per_w - NSLOT + b, b)

    return k(x_flat, lut)


def kernel(x, lut):
    bsz, seq = x.shape
    vocab, d = lut.shape
    x_flat = (
        x.T.reshape(seq // 8, 8, bsz // 128, 128)
        .transpose(0, 2, 1, 3)
        .reshape(-1)
        .astype(jnp.int32)
    )
    out5 = _embed_lookup(x_flat, lut, n_chunks=(bsz * seq) // CHUNK)
    return out5.transpose(2, 4, 0, 1, 3).reshape(bsz, seq, d)


# speedup vs baseline: 1.1628x; 1.0075x over previous
import functools

import jax
import jax.numpy as jnp
from jax import lax
from jax.experimental import pallas as pl
from jax.experimental.pallas import tpu as pltpu
from jax.experimental.pallas import tpu_sc as plsc

D_MODEL = 64
SCALE = 8.0
NUM_WORKERS = 32
CHUNK = 128
NSLOT = 4


@functools.partial(jax.jit, static_argnames=("n_chunks",))
def _embed_lookup(x_flat, lut, *, n_chunks):
    chunks_per_w = n_chunks // NUM_WORKERS
    mesh = plsc.VectorSubcoreMesh(core_axis_name="c", subcore_axis_name="s")

    @functools.partial(
        pl.kernel,
        out_type=jax.ShapeDtypeStruct((200, 8, 128, 8, 128), jnp.float32),
        mesh=mesh,
        scratch_types=[
            [pltpu.VMEM((CHUNK,), jnp.int32) for _ in range(NSLOT)],
            [pltpu.VMEM((CHUNK, D_MODEL), jnp.float32) for _ in range(NSLOT)],
            [pltpu.VMEM((8, 8, CHUNK + 1), jnp.float32) for _ in range(NSLOT)],
            [pltpu.SemaphoreType.DMA for _ in range(NSLOT)],
            [pltpu.SemaphoreType.DMA for _ in range(NSLOT)],
            [pltpu.SemaphoreType.DMA for _ in range(NSLOT)],
        ],
        compiler_params=pltpu.CompilerParams(
            use_tc_tiling_on_sc=False, needs_layout_passes=False
        ),
    )
    def k(x_hbm, lut_hbm, out_hbm, idx, rows, tbuf, isem, gsem, wsem):
        wid = lax.axis_index("s") * 2 + lax.axis_index("c")
        base = wid * chunks_per_w
        lane = lax.broadcasted_iota(jnp.int32, (16,), 0)

        def fetch_idx(i, sl):
            c = base + i
            pltpu.async_copy(x_hbm.at[pl.ds(c * CHUNK, CHUNK)], idx[sl], isem[sl])

        def launch_gather(i, sl):
            c = base + i
            pltpu.make_async_copy(
                x_hbm.at[pl.ds(c * CHUNK, CHUNK)], idx[sl], isem[sl]
            ).wait()
            pltpu.async_copy(lut_hbm.at[idx[sl]], rows[sl], gsem[sl])

        def out_tiles(i, sl):
            c = base + i
            s = (c >> 10) * 8 + (c & 7)
            bt = (c >> 3) & 127
            return [
                (tbuf[sl].at[:, :, pl.ds(0, CHUNK)], out_hbm.at[s, :, bt])
            ]

        def drain_writes(i, sl):
            for src, dst in out_tiles(i, sl):
                pltpu.make_async_copy(src, dst, wsem[sl]).wait()

        def finish(i, sl):
            pltpu.make_async_copy(lut_hbm.at[idx[sl]], rows[sl], gsem[sl]).wait()

            @pl.loop(0, CHUNK, unroll=8)
            def _(r):
                rv = jnp.full((16,), 0, jnp.int32) + r
                for j in range(D_MODEL // 16):
                    v = rows[sl][r, pl.ds(j * 16, 16)]
                    d = j * 16 + lane
                    plsc.store_scatter(
                        tbuf[sl], [d >> 3, d & 7, rv], v * SCALE
                    )

            for src, dst in out_tiles(i, sl):
                pltpu.async_copy(src, dst, wsem[sl])

        fetch_idx(0, 0)
        fetch_idx(1, 1)
        fetch_idx(2, 2)
        launch_gather(0, 0)
        launch_gather(1, 1)

        @pl.loop(0, chunks_per_w, step=NSLOT)
        def _(g):
            for b in range(NSLOT):
                i = g + b
                s2 = (b + 2) % NSLOT
                s3 = (b + 3) % NSLOT

                @pl.when(i + 3 < chunks_per_w)
                def _():
                    fetch_idx(i + 3, s3)

                @pl.when(i + 2 < chunks_per_w)
                def _():
                    launch_gather(i + 2, s2)

                @pl.when(i >= NSLOT)
                def _():
                    drain_writes(i - NSLOT, b)

                finish(i, b)

        for b in range(NSLOT):
            drain_writes(chunks_per_w - NSLOT + b, b)

    return k(x_flat, lut)


def kernel(x, lut):
    bsz, seq = x.shape
    vocab, d = lut.shape
    x_flat = (
        x.T.reshape(seq // 8, 8, bsz // 128, 128)
        .transpose(0, 2, 1, 3)
        .reshape(-1)
        .astype(jnp.int32)
    )
    out5 = _embed_lookup(x_flat, lut, n_chunks=(bsz * seq) // CHUNK)
    return out5.transpose(2, 4, 0, 1, 3).reshape(bsz, seq, d)
